# baseline (speedup 1.0000x reference)
import jax
import jax.numpy as jnp
from jax import lax
from jax.experimental import pallas as pl
from jax.experimental.pallas import tpu as pltpu
from jax.experimental.pallas import tpu_sc as plsc

EPS_ = 1e-5
N_NODES = 10000
N_EDGES = 320000
ND = 128
HD = 16
PACK = 8
EPR = N_EDGES // PACK

NW = 32
CH = 128
GJ = 20
NG = 4
CPW = GJ * NG
NCH = NW * CPW
E_PAD = NCH * CH
EP_PAD = E_PAD // PACK
GROUP_E = GJ * CH
GROUP_PR = GROUP_E // PACK
WPR = CPW * CH // PACK
NPT = N_NODES // 16

_sc_mesh = plsc.VectorSubcoreMesh(core_axis_name="c", subcore_axis_name="s")
_sc_params = pltpu.CompilerParams(use_tc_tiling_on_sc=False)


def _sc_gather_body(xs_hbm, xr_hbm, s2_hbm, r2_hbm, ea_hbm, zp_hbm,
                    gs_hbm, gr_hbm, eap_hbm, sidx, ridx, rows, sem):
    cid = lax.axis_index("c")
    sid = lax.axis_index("s")
    w = sid * 2 + cid
    base_c = w * CPW
    base_e = w * CPW * CH
    @pl.when(w < NW - 1)
    def _():
        cpa = pltpu.async_copy(ea_hbm.at[pl.ds(base_e, CPW * CH)],
                               eap_hbm.at[pl.ds(base_e, CPW * CH)], sem)
        cpa.wait()

    @pl.when(w == NW - 1)
    def _():
        real = N_EDGES - (NW - 1) * CPW * CH
        cpa = pltpu.async_copy(ea_hbm.at[pl.ds(base_e, real)],
                               eap_hbm.at[pl.ds(base_e, real)], sem)
        cpb = pltpu.async_copy(zp_hbm,
                               eap_hbm.at[pl.ds(N_EDGES, E_PAD - N_EDGES)], sem)
        cpa.wait()
        cpb.wait()

    pltpu.sync_copy(s2_hbm.at[pl.ds(base_c, CPW)], sidx)
    pltpu.sync_copy(r2_hbm.at[pl.ds(base_c, CPW)], ridx)

    def do_table(tab_hbm, idx, out_hbm):
        def body(g, carry):
            cps = [pltpu.async_copy(tab_hbm.at[idx.at[g * GJ + j]],
                                    rows.at[pl.ds(j * CH, CH)], sem)
                   for j in range(GJ)]
            for cp in cps:
                cp.wait()
            pltpu.sync_copy(rows,
                            out_hbm.at[pl.ds(w * CPW * CH + g * GROUP_E,
                                             GROUP_E)])
            return carry
        lax.fori_loop(0, NG, body, 0)

    do_table(xs_hbm, sidx, gs_hbm)
    do_table(xr_hbm, ridx, gr_hbm)


_gather_call = pl.kernel(
    _sc_gather_body,
    out_type=[jax.ShapeDtypeStruct((E_PAD, HD), jnp.float32),
              jax.ShapeDtypeStruct((E_PAD, HD), jnp.float32),
              jax.ShapeDtypeStruct((E_PAD, HD), jnp.float32)],
    mesh=_sc_mesh,
    scratch_types=[pltpu.VMEM((CPW, CH), jnp.int32),
                   pltpu.VMEM((CPW, CH), jnp.int32),
                   pltpu.VMEM((GROUP_E, HD), jnp.float32),
                   pltpu.SemaphoreType.DMA],
    compiler_params=_sc_params,
)


def _sc_scatter_body(vals_hbm, r2_hbm, z_hbm, out_hbm, ean_hbm, idx, rows, acc):
    cid = lax.axis_index("c")
    sid = lax.axis_index("s")
    w = sid * 2 + cid
    base_c = w * CPW
    pltpu.sync_copy(z_hbm.at[pl.ds(sid * NPT, NPT)],
                    acc.at[pl.ds(sid * NPT, NPT)])
    pltpu.sync_copy(r2_hbm.at[pl.ds(base_c, CPW)], idx)
    plsc.subcore_barrier()

    def body(g, carry):
        base_e = w * CPW * CH + g * GROUP_E
        pltpu.sync_copy(vals_hbm.at[pl.ds(base_e, GROUP_E)], rows)
        @pl.when(base_e < N_EDGES)
        def _():
            pltpu.sync_copy(rows, ean_hbm.at[pl.ds(base_e, GROUP_E)])
        for j in range(GJ):
            pltpu.sync_copy(rows.at[pl.ds(j * CH, CH)],
                            acc.at[idx.at[g * GJ + j]], add=True)
        return carry
    lax.fori_loop(0, NG, body, 0)

    plsc.subcore_barrier()
    pltpu.sync_copy(acc.at[pl.ds(sid * NPT, NPT)],
                    out_hbm.at[cid, pl.ds(sid * NPT, NPT)])


_scatter_call = pl.kernel(
    _sc_scatter_body,
    out_type=[jax.ShapeDtypeStruct((2, N_NODES, HD), jnp.float32),
              jax.ShapeDtypeStruct((N_EDGES, HD), jnp.float32)],
    mesh=_sc_mesh,
    scratch_types=[pltpu.VMEM((CPW, CH), jnp.int32),
                   pltpu.VMEM((GROUP_E, HD), jnp.float32),
                   pltpu.VMEM_SHARED((N_NODES, HD), jnp.float32)],
    compiler_params=_sc_params,
)


def _proj_body(x_ref, wa_ref, wb_ref, oa_ref, ob_ref):
    xv = x_ref[...]
    oa_ref[...] = jnp.dot(xv, wa_ref[...], preferred_element_type=jnp.float32)
    ob_ref[...] = jnp.dot(xv, wb_ref[...], preferred_element_type=jnp.float32)


def _edge_body(gs_ref, gr_ref, ea_ref, w1_ref, w2_ref, gm_ref,
               b1_ref, b2_ref, g_ref, bt_ref, o_ref):
    bidx = pl.program_id(0)
    eav = ea_ref[...]
    pre = (gs_ref[...] + gr_ref[...]
           + jnp.dot(eav, w1_ref[...], preferred_element_type=jnp.float32)
           + b1_ref[...])
    h1 = jnp.maximum(pre, 0.0)
    h = jnp.dot(h1, w2_ref[...], preferred_element_type=jnp.float32) + b2_ref[...]
    mu = jnp.dot(h, gm_ref[...], preferred_element_type=jnp.float32)
    d = h - mu
    var = jnp.dot(d * d, gm_ref[...], preferred_element_type=jnp.float32)
    res = eav + d * lax.rsqrt(var + EPS_) * g_ref[...] + bt_ref[...]
    row = bidx * BBLK + lax.broadcasted_iota(jnp.int32, res.shape, 0)
    o_ref[...] = jnp.where(row < EPR, res, 0.0)


def _node_body(x_ref, p0_ref, p1_ref, w1a_ref, w1b_ref, w2_ref,
               b1_ref, b2_ref, g_ref, bt_ref, o_ref):
    xv = x_ref[...]
    agg = p0_ref[...] + p1_ref[...]
    h1 = jnp.maximum(
        jnp.dot(xv, w1a_ref[...], preferred_element_type=jnp.float32)
        + jnp.dot(agg, w1b_ref[...], preferred_element_type=jnp.float32)
        + b1_ref[...], 0.0)
    u = jnp.dot(h1, w2_ref[...], preferred_element_type=jnp.float32) + b2_ref[...]
    mu = jnp.mean(u, axis=-1, keepdims=True)
    d = u - mu
    var = jnp.mean(d * d, axis=-1, keepdims=True)
    o_ref[...] = xv + d * lax.rsqrt(var + EPS_) * g_ref[...] + bt_ref[...]


BBLK = 2048


def kernel(x, edge_index, edge_attr, eW1, eb1, eW2, eb2,
           nW1, nb1, nW2, nb2, eg, ebt, ng, nbt):
    f32 = jnp.float32
    pad1 = jnp.zeros((E_PAD - N_EDGES,), jnp.int32)
    s2 = jnp.concatenate([edge_index[0], pad1]).reshape(NCH, CH)
    r2 = jnp.concatenate([edge_index[1], pad1]).reshape(NCH, CH)

    xs, xr = pl.pallas_call(
        _proj_body,
        grid=(10,),
        in_specs=[pl.BlockSpec((1000, ND), lambda i: (i, 0)),
                  pl.BlockSpec((ND, HD), lambda i: (0, 0)),
                  pl.BlockSpec((ND, HD), lambda i: (0, 0))],
        out_specs=[pl.BlockSpec((1000, HD), lambda i: (i, 0))] * 2,
        out_shape=[jax.ShapeDtypeStruct((N_NODES, HD), f32)] * 2,
    )(x, eW1[:ND], eW1[ND:2 * ND])

    gs, gr, ea_pad = _gather_call(xs, xr, s2, r2, edge_attr,
                                  jnp.zeros((E_PAD - N_EDGES, HD), f32))
    gs_p = gs.reshape(EP_PAD, ND)
    gr_p = gr.reshape(EP_PAD, ND)

    eye8 = jnp.eye(PACK, dtype=f32)
    w1bd = jnp.kron(eye8, eW1[2 * ND:])
    w2bd = jnp.kron(eye8, eW2)
    gmat = jnp.kron(eye8, jnp.full((HD, HD), 1.0 / HD, f32))
    b1t = jnp.tile(eb1, PACK).reshape(1, ND)
    b2t = jnp.tile(eb2, PACK).reshape(1, ND)
    egt = jnp.tile(eg, PACK).reshape(1, ND)
    ebtt = jnp.tile(ebt, PACK).reshape(1, ND)

    ea_p = ea_pad.reshape(EP_PAD, ND)

    ean_p = pl.pallas_call(
        _edge_body,
        grid=(EP_PAD // BBLK,),
        in_specs=[pl.BlockSpec((BBLK, ND), lambda i: (i, 0))] * 3
        + [pl.BlockSpec((ND, ND), lambda i: (0, 0))] * 3
        + [pl.BlockSpec((1, ND), lambda i: (0, 0))] * 4,
        out_specs=pl.BlockSpec((BBLK, ND), lambda i: (i, 0)),
        out_shape=jax.ShapeDtypeStruct((EP_PAD, ND), f32),
    )(gs_p, gr_p, ea_p, w1bd, w2bd, gmat, b1t, b2t, egt, ebtt)

    parts, edge_attr_new = _scatter_call(ean_p.reshape(E_PAD, HD), r2,
                                         jnp.zeros((N_NODES, HD), f32))

    x_new = pl.pallas_call(
        _node_body,
        grid=(10,),
        in_specs=[pl.BlockSpec((1000, ND), lambda i: (i, 0)),
                  pl.BlockSpec((1000, HD), lambda i: (i, 0)),
                  pl.BlockSpec((1000, HD), lambda i: (i, 0)),
                  pl.BlockSpec((ND, HD), lambda i: (0, 0)),
                  pl.BlockSpec((HD, HD), lambda i: (0, 0)),
                  pl.BlockSpec((HD, ND), lambda i: (0, 0)),
                  pl.BlockSpec((1, HD), lambda i: (0, 0)),
                  pl.BlockSpec((1, ND), lambda i: (0, 0)),
                  pl.BlockSpec((1, ND), lambda i: (0, 0)),
                  pl.BlockSpec((1, ND), lambda i: (0, 0))],
        out_specs=pl.BlockSpec((1000, ND), lambda i: (i, 0)),
        out_shape=jax.ShapeDtypeStruct((N_NODES, ND), f32),
    )(x, parts[0], parts[1], nW1[:ND], nW1[ND:], nW2,
      nb1.reshape(1, HD), nb2.reshape(1, ND), ng.reshape(1, ND),
      nbt.reshape(1, ND))

    return (x_new, edge_attr_new)


# speedup vs baseline: 1.9047x; 1.9047x over previous
import jax
import jax.numpy as jnp
from jax import lax
from jax.experimental import pallas as pl
from jax.experimental.pallas import tpu as pltpu
from jax.experimental.pallas import tpu_sc as plsc

EPS_ = 1e-5
N_NODES = 10000
N_EDGES = 320000
ND = 128
HD = 16
PACK = 8
EPR = N_EDGES // PACK

NW = 32
CH = 128
GJ = 20
NG = 4
CPW = GJ * NG
NCH = NW * CPW
E_PAD = NCH * CH
EP_PAD = E_PAD // PACK
GROUP_E = GJ * CH
GROUP_PR = GROUP_E // PACK
WPR = CPW * CH // PACK
NPT = N_NODES // 16

_sc_mesh = plsc.VectorSubcoreMesh(core_axis_name="c", subcore_axis_name="s")
_sc_params = pltpu.CompilerParams(use_tc_tiling_on_sc=False)


def _sc_gather_body(xs_hbm, xr_hbm, s2_hbm, r2_hbm, ea_hbm, zp_hbm,
                    gs_hbm, gr_hbm, eap_hbm, sidx, ridx, rows, sem):
    cid = lax.axis_index("c")
    sid = lax.axis_index("s")
    w = sid * 2 + cid
    base_c = w * CPW
    def ea_body(g, carry):
        base_e = w * CPW * CH + g * GROUP_E

        @pl.when(base_e < N_EDGES)
        def _():
            pltpu.sync_copy(ea_hbm.at[pl.ds(base_e, GROUP_E)], rows)

        @pl.when(base_e >= N_EDGES)
        def _():
            pltpu.sync_copy(zp_hbm.at[pl.ds(base_e - N_EDGES, GROUP_E)], rows)

        pltpu.sync_copy(rows, eap_hbm.at[pl.ds(base_e, GROUP_E)])
        return carry
    lax.fori_loop(0, NG, ea_body, 0)

    pltpu.sync_copy(s2_hbm.at[pl.ds(base_c, CPW)], sidx)
    pltpu.sync_copy(r2_hbm.at[pl.ds(base_c, CPW)], ridx)

    def do_table(tab_hbm, idx, out_hbm):
        def body(g, carry):
            cps = [pltpu.async_copy(tab_hbm.at[idx.at[g * GJ + j]],
                                    rows.at[pl.ds(j * CH, CH)], sem)
                   for j in range(GJ)]
            for cp in cps:
                cp.wait()
            pltpu.sync_copy(rows,
                            out_hbm.at[pl.ds(w * CPW * CH + g * GROUP_E,
                                             GROUP_E)])
            return carry
        lax.fori_loop(0, NG, body, 0)

    do_table(xs_hbm, sidx, gs_hbm)
    do_table(xr_hbm, ridx, gr_hbm)


_gather_call = pl.kernel(
    _sc_gather_body,
    out_type=[jax.ShapeDtypeStruct((E_PAD, HD), jnp.float32),
              jax.ShapeDtypeStruct((E_PAD, HD), jnp.float32),
              jax.ShapeDtypeStruct((E_PAD, HD), jnp.float32)],
    mesh=_sc_mesh,
    scratch_types=[pltpu.VMEM((CPW, CH), jnp.int32),
                   pltpu.VMEM((CPW, CH), jnp.int32),
                   pltpu.VMEM((GROUP_E, HD), jnp.float32),
                   pltpu.SemaphoreType.DMA],
    compiler_params=_sc_params,
)


def _sc_scatter_body(vals_hbm, r2_hbm, z_hbm, out_hbm, ean_hbm, idx, rows, acc):
    cid = lax.axis_index("c")
    sid = lax.axis_index("s")
    w = sid * 2 + cid
    base_c = w * CPW
    pltpu.sync_copy(z_hbm.at[pl.ds(sid * NPT, NPT)],
                    acc.at[pl.ds(sid * NPT, NPT)])
    pltpu.sync_copy(r2_hbm.at[pl.ds(base_c, CPW)], idx)
    plsc.subcore_barrier()

    def body(g, carry):
        base_e = w * CPW * CH + g * GROUP_E
        pltpu.sync_copy(vals_hbm.at[pl.ds(base_e, GROUP_E)], rows)
        @pl.when(base_e < N_EDGES)
        def _():
            pltpu.sync_copy(rows, ean_hbm.at[pl.ds(base_e, GROUP_E)])
        for j in range(GJ):
            pltpu.sync_copy(rows.at[pl.ds(j * CH, CH)],
                            acc.at[idx.at[g * GJ + j]], add=True)
        return carry
    lax.fori_loop(0, NG, body, 0)

    plsc.subcore_barrier()
    pltpu.sync_copy(acc.at[pl.ds(sid * NPT, NPT)],
                    out_hbm.at[cid, pl.ds(sid * NPT, NPT)])


_scatter_call = pl.kernel(
    _sc_scatter_body,
    out_type=[jax.ShapeDtypeStruct((2, N_NODES, HD), jnp.float32),
              jax.ShapeDtypeStruct((N_EDGES, HD), jnp.float32)],
    mesh=_sc_mesh,
    scratch_types=[pltpu.VMEM((CPW, CH), jnp.int32),
                   pltpu.VMEM((GROUP_E, HD), jnp.float32),
                   pltpu.VMEM_SHARED((N_NODES, HD), jnp.float32)],
    compiler_params=_sc_params,
)


def _proj_body(x_ref, wa_ref, wb_ref, oa_ref, ob_ref):
    xv = x_ref[...]
    oa_ref[...] = jnp.dot(xv, wa_ref[...], preferred_element_type=jnp.float32)
    ob_ref[...] = jnp.dot(xv, wb_ref[...], preferred_element_type=jnp.float32)


def _edge_body(gs_ref, gr_ref, ea_ref, w1_ref, w2_ref, gm_ref,
               b1_ref, b2_ref, g_ref, bt_ref, o_ref):
    bidx = pl.program_id(0)
    eav = ea_ref[...]
    pre = (gs_ref[...] + gr_ref[...]
           + jnp.dot(eav, w1_ref[...], preferred_element_type=jnp.float32)
           + b1_ref[...])
    h1 = jnp.maximum(pre, 0.0)
    h = jnp.dot(h1, w2_ref[...], preferred_element_type=jnp.float32) + b2_ref[...]
    mu = jnp.dot(h, gm_ref[...], preferred_element_type=jnp.float32)
    d = h - mu
    var = jnp.dot(d * d, gm_ref[...], preferred_element_type=jnp.float32)
    res = eav + d * lax.rsqrt(var + EPS_) * g_ref[...] + bt_ref[...]
    row = bidx * BBLK + lax.broadcasted_iota(jnp.int32, res.shape, 0)
    o_ref[...] = jnp.where(row < EPR, res, 0.0)


def _node_body(x_ref, p0_ref, p1_ref, w1a_ref, w1b_ref, w2_ref,
               b1_ref, b2_ref, g_ref, bt_ref, o_ref):
    xv = x_ref[...]
    agg = p0_ref[...] + p1_ref[...]
    h1 = jnp.maximum(
        jnp.dot(xv, w1a_ref[...], preferred_element_type=jnp.float32)
        + jnp.dot(agg, w1b_ref[...], preferred_element_type=jnp.float32)
        + b1_ref[...], 0.0)
    u = jnp.dot(h1, w2_ref[...], preferred_element_type=jnp.float32) + b2_ref[...]
    mu = jnp.mean(u, axis=-1, keepdims=True)
    d = u - mu
    var = jnp.mean(d * d, axis=-1, keepdims=True)
    o_ref[...] = xv + d * lax.rsqrt(var + EPS_) * g_ref[...] + bt_ref[...]


BBLK = 2048


def kernel(x, edge_index, edge_attr, eW1, eb1, eW2, eb2,
           nW1, nb1, nW2, nb2, eg, ebt, ng, nbt):
    f32 = jnp.float32
    pad1 = jnp.zeros((E_PAD - N_EDGES,), jnp.int32)
    s2 = jnp.concatenate([edge_index[0], pad1]).reshape(NCH, CH)
    r2 = jnp.concatenate([edge_index[1], pad1]).reshape(NCH, CH)

    xs, xr = pl.pallas_call(
        _proj_body,
        grid=(10,),
        in_specs=[pl.BlockSpec((1000, ND), lambda i: (i, 0)),
                  pl.BlockSpec((ND, HD), lambda i: (0, 0)),
                  pl.BlockSpec((ND, HD), lambda i: (0, 0))],
        out_specs=[pl.BlockSpec((1000, HD), lambda i: (i, 0))] * 2,
        out_shape=[jax.ShapeDtypeStruct((N_NODES, HD), f32)] * 2,
    )(x, eW1[:ND], eW1[ND:2 * ND])

    gs, gr, ea_pad = _gather_call(xs, xr, s2, r2, edge_attr,
                                  jnp.zeros((E_PAD - N_EDGES, HD), f32))
    gs_p = gs.reshape(EP_PAD, ND)
    gr_p = gr.reshape(EP_PAD, ND)

    eye8 = jnp.eye(PACK, dtype=f32)
    w1bd = jnp.kron(eye8, eW1[2 * ND:])
    w2bd = jnp.kron(eye8, eW2)
    gmat = jnp.kron(eye8, jnp.full((HD, HD), 1.0 / HD, f32))
    b1t = jnp.tile(eb1, PACK).reshape(1, ND)
    b2t = jnp.tile(eb2, PACK).reshape(1, ND)
    egt = jnp.tile(eg, PACK).reshape(1, ND)
    ebtt = jnp.tile(ebt, PACK).reshape(1, ND)

    ea_p = ea_pad.reshape(EP_PAD, ND)

    ean_p = pl.pallas_call(
        _edge_body,
        grid=(EP_PAD // BBLK,),
        in_specs=[pl.BlockSpec((BBLK, ND), lambda i: (i, 0))] * 3
        + [pl.BlockSpec((ND, ND), lambda i: (0, 0))] * 3
        + [pl.BlockSpec((1, ND), lambda i: (0, 0))] * 4,
        out_specs=pl.BlockSpec((BBLK, ND), lambda i: (i, 0)),
        out_shape=jax.ShapeDtypeStruct((EP_PAD, ND), f32),
    )(gs_p, gr_p, ea_p, w1bd, w2bd, gmat, b1t, b2t, egt, ebtt)

    parts, edge_attr_new = _scatter_call(ean_p.reshape(E_PAD, HD), r2,
                                         jnp.zeros((N_NODES, HD), f32))

    x_new = pl.pallas_call(
        _node_body,
        grid=(10,),
        in_specs=[pl.BlockSpec((1000, ND), lambda i: (i, 0)),
                  pl.BlockSpec((1000, HD), lambda i: (i, 0)),
                  pl.BlockSpec((1000, HD), lambda i: (i, 0)),
                  pl.BlockSpec((ND, HD), lambda i: (0, 0)),
                  pl.BlockSpec((HD, HD), lambda i: (0, 0)),
                  pl.BlockSpec((HD, ND), lambda i: (0, 0)),
                  pl.BlockSpec((1, HD), lambda i: (0, 0)),
                  pl.BlockSpec((1, ND), lambda i: (0, 0)),
                  pl.BlockSpec((1, ND), lambda i: (0, 0)),
                  pl.BlockSpec((1, ND), lambda i: (0, 0))],
        out_specs=pl.BlockSpec((1000, ND), lambda i: (i, 0)),
        out_shape=jax.ShapeDtypeStruct((N_NODES, ND), f32),
    )(x, parts[0], parts[1], nW1[:ND], nW1[ND:], nW2,
      nb1.reshape(1, HD), nb2.reshape(1, ND), ng.reshape(1, ND),
      nbt.reshape(1, ND))

    return (x_new, edge_attr_new)


# speedup vs baseline: 2.7435x; 1.4404x over previous
import jax
import jax.numpy as jnp
from jax import lax
from jax.experimental import pallas as pl
from jax.experimental.pallas import tpu as pltpu
from jax.experimental.pallas import tpu_sc as plsc

EPS_ = 1e-5
N_NODES = 10000
N_EDGES = 320000
ND = 128
HD = 16
PACK = 8
EPR = N_EDGES // PACK

NW = 32
CH = 128
GJ = 20
GROUP_E = GJ * CH
NGRP = N_EDGES // GROUP_E
NCH = N_EDGES // CH
S2R = 2520
MAXG = 4
NPT = N_NODES // 16

_sc_mesh = plsc.VectorSubcoreMesh(core_axis_name="c", subcore_axis_name="s")
_sc_params = pltpu.CompilerParams(use_tc_tiling_on_sc=False)


def _worker_span(w):
    cnt = jnp.where(w < 29, 4, 3)
    base = jnp.where(w < 29, 4 * w, 116 + 3 * (w - 29))
    return base, cnt


def _sc_gather_body(xs_hbm, xr_hbm, s2_hbm, r2_hbm, gs_hbm, gr_hbm,
                    sidx, ridx, rows, sem):
    cid = lax.axis_index("c")
    sid = lax.axis_index("s")
    w = sid * 2 + cid
    gbase, gcnt = _worker_span(w)
    base_c = gbase * GJ
    pltpu.sync_copy(s2_hbm.at[pl.ds(base_c, MAXG * GJ)], sidx)
    pltpu.sync_copy(r2_hbm.at[pl.ds(base_c, MAXG * GJ)], ridx)

    def do_table(tab_hbm, idx, out_hbm):
        def body(g, carry):
            cps = [pltpu.async_copy(tab_hbm.at[idx.at[g * GJ + j]],
                                    rows.at[pl.ds(j * CH, CH)], sem)
                   for j in range(GJ)]
            for cp in cps:
                cp.wait()
            pltpu.sync_copy(rows,
                            out_hbm.at[pl.ds((gbase + g) * GROUP_E, GROUP_E)])
            return carry
        lax.fori_loop(0, gcnt, body, 0)

    do_table(xs_hbm, sidx, gs_hbm)
    do_table(xr_hbm, ridx, gr_hbm)


_gather_call = pl.kernel(
    _sc_gather_body,
    out_type=[jax.ShapeDtypeStruct((N_EDGES, HD), jnp.float32),
              jax.ShapeDtypeStruct((N_EDGES, HD), jnp.float32)],
    mesh=_sc_mesh,
    scratch_types=[pltpu.VMEM((MAXG * GJ, CH), jnp.int32),
                   pltpu.VMEM((MAXG * GJ, CH), jnp.int32),
                   pltpu.VMEM((GROUP_E, HD), jnp.float32),
                   pltpu.SemaphoreType.DMA],
    compiler_params=_sc_params,
)


def _sc_scatter_body(vals_hbm, r2_hbm, z_hbm, out_hbm, idx, rows, acc):
    cid = lax.axis_index("c")
    sid = lax.axis_index("s")
    w = sid * 2 + cid
    gbase, gcnt = _worker_span(w)
    base_c = gbase * GJ
    pltpu.sync_copy(z_hbm.at[pl.ds(sid * NPT, NPT)],
                    acc.at[pl.ds(sid * NPT, NPT)])
    pltpu.sync_copy(r2_hbm.at[pl.ds(base_c, MAXG * GJ)], idx)
    plsc.subcore_barrier()

    def body(g, carry):
        pltpu.sync_copy(vals_hbm.at[pl.ds((gbase + g) * GROUP_E, GROUP_E)],
                        rows)
        for j in range(GJ):
            pltpu.sync_copy(rows.at[pl.ds(j * CH, CH)],
                            acc.at[idx.at[g * GJ + j]], add=True)
        return carry
    lax.fori_loop(0, gcnt, body, 0)

    plsc.subcore_barrier()
    pltpu.sync_copy(acc.at[pl.ds(sid * NPT, NPT)],
                    out_hbm.at[cid, pl.ds(sid * NPT, NPT)])


_scatter_call = pl.kernel(
    _sc_scatter_body,
    out_type=jax.ShapeDtypeStruct((2, N_NODES, HD), jnp.float32),
    mesh=_sc_mesh,
    scratch_types=[pltpu.VMEM((MAXG * GJ, CH), jnp.int32),
                   pltpu.VMEM((GROUP_E, HD), jnp.float32),
                   pltpu.VMEM_SHARED((N_NODES, HD), jnp.float32)],
    compiler_params=_sc_params,
)


def _proj_body(x_ref, wa_ref, wb_ref, oa_ref, ob_ref):
    xv = x_ref[...]
    oa_ref[...] = jnp.dot(xv, wa_ref[...], preferred_element_type=jnp.float32)
    ob_ref[...] = jnp.dot(xv, wb_ref[...], preferred_element_type=jnp.float32)


def _edge_body(gs_ref, gr_ref, ea_ref, w1_ref, w2_ref, gm_ref,
               b1_ref, b2_ref, g_ref, bt_ref, o_ref):
    eav = ea_ref[...]
    pre = (gs_ref[...] + gr_ref[...]
           + jnp.dot(eav, w1_ref[...], preferred_element_type=jnp.float32)
           + b1_ref[...])
    h1 = jnp.maximum(pre, 0.0)
    h = jnp.dot(h1, w2_ref[...], preferred_element_type=jnp.float32) + b2_ref[...]
    mu = jnp.dot(h, gm_ref[...], preferred_element_type=jnp.float32)
    d = h - mu
    var = jnp.dot(d * d, gm_ref[...], preferred_element_type=jnp.float32)
    o_ref[...] = eav + d * lax.rsqrt(var + EPS_) * g_ref[...] + bt_ref[...]


def _node_body(x_ref, p0_ref, p1_ref, w1a_ref, w1b_ref, w2_ref,
               b1_ref, b2_ref, g_ref, bt_ref, o_ref):
    xv = x_ref[...]
    agg = p0_ref[...] + p1_ref[...]
    h1 = jnp.maximum(
        jnp.dot(xv, w1a_ref[...], preferred_element_type=jnp.float32)
        + jnp.dot(agg, w1b_ref[...], preferred_element_type=jnp.float32)
        + b1_ref[...], 0.0)
    u = jnp.dot(h1, w2_ref[...], preferred_element_type=jnp.float32) + b2_ref[...]
    mu = jnp.mean(u, axis=-1, keepdims=True)
    d = u - mu
    var = jnp.mean(d * d, axis=-1, keepdims=True)
    o_ref[...] = xv + d * lax.rsqrt(var + EPS_) * g_ref[...] + bt_ref[...]


BBLK = 2000


def kernel(x, edge_index, edge_attr, eW1, eb1, eW2, eb2,
           nW1, nb1, nW2, nb2, eg, ebt, ng, nbt):
    f32 = jnp.float32
    pad1 = jnp.zeros((S2R * CH - N_EDGES,), jnp.int32)
    s2 = jnp.concatenate([edge_index[0], pad1]).reshape(S2R, CH)
    r2 = jnp.concatenate([edge_index[1], pad1]).reshape(S2R, CH)

    xs, xr = pl.pallas_call(
        _proj_body,
        grid=(10,),
        in_specs=[pl.BlockSpec((1000, ND), lambda i: (i, 0)),
                  pl.BlockSpec((ND, HD), lambda i: (0, 0)),
                  pl.BlockSpec((ND, HD), lambda i: (0, 0))],
        out_specs=[pl.BlockSpec((1000, HD), lambda i: (i, 0))] * 2,
        out_shape=[jax.ShapeDtypeStruct((N_NODES, HD), f32)] * 2,
    )(x, eW1[:ND], eW1[ND:2 * ND])

    gs, gr = _gather_call(xs, xr, s2, r2)

    eye8 = jnp.eye(PACK, dtype=f32)
    w1bd = jnp.kron(eye8, eW1[2 * ND:])
    w2bd = jnp.kron(eye8, eW2)
    gmat = jnp.kron(eye8, jnp.full((HD, HD), 1.0 / HD, f32))
    b1t = jnp.tile(eb1, PACK).reshape(1, ND)
    b2t = jnp.tile(eb2, PACK).reshape(1, ND)
    egt = jnp.tile(eg, PACK).reshape(1, ND)
    ebtt = jnp.tile(ebt, PACK).reshape(1, ND)

    ean_p = pl.pallas_call(
        _edge_body,
        grid=(EPR // BBLK,),
        in_specs=[pl.BlockSpec((BBLK, ND), lambda i: (i, 0))] * 3
        + [pl.BlockSpec((ND, ND), lambda i: (0, 0))] * 3
        + [pl.BlockSpec((1, ND), lambda i: (0, 0))] * 4,
        out_specs=pl.BlockSpec((BBLK, ND), lambda i: (i, 0)),
        out_shape=jax.ShapeDtypeStruct((EPR, ND), f32),
    )(gs.reshape(EPR, ND), gr.reshape(EPR, ND), edge_attr.reshape(EPR, ND),
      w1bd, w2bd, gmat, b1t, b2t, egt, ebtt)

    edge_attr_new = ean_p.reshape(N_EDGES, HD)

    parts = _scatter_call(edge_attr_new, r2, jnp.zeros((N_NODES, HD), f32))

    x_new = pl.pallas_call(
        _node_body,
        grid=(10,),
        in_specs=[pl.BlockSpec((1000, ND), lambda i: (i, 0)),
                  pl.BlockSpec((1000, HD), lambda i: (i, 0)),
                  pl.BlockSpec((1000, HD), lambda i: (i, 0)),
                  pl.BlockSpec((ND, HD), lambda i: (0, 0)),
                  pl.BlockSpec((HD, HD), lambda i: (0, 0)),
                  pl.BlockSpec((HD, ND), lambda i: (0, 0)),
                  pl.BlockSpec((1, HD), lambda i: (0, 0)),
                  pl.BlockSpec((1, ND), lambda i: (0, 0)),
                  pl.BlockSpec((1, ND), lambda i: (0, 0)),
                  pl.BlockSpec((1, ND), lambda i: (0, 0))],
        out_specs=pl.BlockSpec((1000, ND), lambda i: (i, 0)),
        out_shape=jax.ShapeDtypeStruct((N_NODES, ND), f32),
    )(x, parts[0], parts[1], nW1[:ND], nW1[ND:], nW2,
      nb1.reshape(1, HD), nb2.reshape(1, ND), ng.reshape(1, ND),
      nbt.reshape(1, ND))

    return (x_new, edge_attr_new)
